# Initial kernel scaffold; baseline (speedup 1.0000x reference)
import jax
import jax.numpy as jnp
from jax.experimental import pallas as pl


def kernel(input_ids, position_ids, word_emb, pos_emb, ln_w, ln_b):
    raise NotImplementedError("write your pallas kernel here")


# speedup vs baseline: 1.3355x; 1.3355x over previous
import functools

import jax
import jax.numpy as jnp
from jax import lax
from jax.experimental import pallas as pl
from jax.experimental.pallas import tpu as pltpu
from jax.experimental.pallas import tpu_sc as plsc

VOCAB = 100000
HIDDEN = 128
MAX_POS = 512
EPS = 1e-12
L = 16
NV = HIDDEN // L
NC, NS = 2, 16
NW = NC * NS
N_TOK = 64 * 32 * 32
CHUNK = 128
PER_W = N_TOK // NW
N_CHUNKS = PER_W // CHUNK


def _rsqrt_nr(xv):
    iv = lax.bitcast_convert_type(xv, jnp.int32)
    iv = jnp.int32(0x5F3759DF) - (iv >> 1)
    y = lax.bitcast_convert_type(iv, jnp.float32)
    for _ in range(3):
        y = y * (1.5 - 0.5 * xv * y * y)
    return y


def _sc_embed_ln(ids_hbm, pids_hbm, wemb_hbm, pemb_hbm, lnw_hbm, lnb_hbm,
                 out_hbm, idw_v, idp_v, wrows_v, prows_v, lnw_v, lnb_v,
                 semw, semp):
    wid = lax.axis_index("s") * NC + lax.axis_index("c")
    base_w = wid * PER_W

    pltpu.sync_copy(lnw_hbm, lnw_v)
    pltpu.sync_copy(lnb_hbm, lnb_v)
    lnw = [lnw_v[pl.ds(L * j, L)] for j in range(NV)]
    lnb = [lnb_v[pl.ds(L * j, L)] for j in range(NV)]

    def token_body(t, _):
        w = [wrows_v[t, pl.ds(L * j, L)] + prows_v[t, pl.ds(L * j, L)]
             for j in range(NV)]
        s = ((w[0] + w[1]) + (w[2] + w[3])) + ((w[4] + w[5]) + (w[6] + w[7]))
        u = jnp.sum(s) * (1.0 / HIDDEN)
        uv = jnp.full((L,), u, jnp.float32)
        d = [wj - uv for wj in w]
        sq = (((d[0] * d[0] + d[1] * d[1]) + (d[2] * d[2] + d[3] * d[3]))
              + ((d[4] * d[4] + d[5] * d[5]) + (d[6] * d[6] + d[7] * d[7])))
        var = jnp.sum(sq) * (1.0 / HIDDEN)
        xv = jnp.full((L,), var + EPS, jnp.float32)
        r = _rsqrt_nr(xv)
        for j in range(NV):
            wrows_v[t, pl.ds(L * j, L)] = d[j] * r * lnw[j] + lnb[j]
        return _

    for c in range(N_CHUNKS):
        base = base_w + c * CHUNK
        pltpu.sync_copy(ids_hbm.at[pl.ds(base, CHUNK)], idw_v)
        pltpu.sync_copy(pids_hbm.at[pl.ds(base, CHUNK)], idp_v)
        cw = pltpu.async_copy(wemb_hbm.at[idw_v], wrows_v, semw)
        cp = pltpu.async_copy(pemb_hbm.at[idp_v], prows_v, semp)
        cw.wait()
        cp.wait()
        lax.fori_loop(0, CHUNK, token_body, None)
        pltpu.sync_copy(wrows_v, out_hbm.at[pl.ds(base, CHUNK)])


@jax.jit
def kernel(input_ids, position_ids, word_emb, pos_emb, ln_w, ln_b):
    ids = input_ids.reshape(-1)
    pids = position_ids.reshape(-1)
    mesh = plsc.VectorSubcoreMesh(core_axis_name="c", subcore_axis_name="s")
    k = functools.partial(
        pl.kernel,
        mesh=mesh,
        compiler_params=pltpu.CompilerParams(needs_layout_passes=False),
        out_type=jax.ShapeDtypeStruct((N_TOK, HIDDEN), jnp.float32),
        scratch_types=[
            pltpu.VMEM((CHUNK,), jnp.int32),
            pltpu.VMEM((CHUNK,), jnp.int32),
            pltpu.VMEM((CHUNK, HIDDEN), jnp.float32),
            pltpu.VMEM((CHUNK, HIDDEN), jnp.float32),
            pltpu.VMEM((HIDDEN,), jnp.float32),
            pltpu.VMEM((HIDDEN,), jnp.float32),
            pltpu.SemaphoreType.DMA,
            pltpu.SemaphoreType.DMA,
        ],
    )(_sc_embed_ln)
    out = k(ids, pids, word_emb, pos_emb, ln_w, ln_b)
    return out.reshape(input_ids.shape + (HIDDEN,))


# speedup vs baseline: 3.4984x; 2.6195x over previous
import functools

import jax
import jax.numpy as jnp
from jax import lax
from jax.experimental import pallas as pl
from jax.experimental.pallas import tpu as pltpu
from jax.experimental.pallas import tpu_sc as plsc

HIDDEN = 128
EPS = 1e-12
L = 16
NV = HIDDEN // L
NC, NS = 2, 16
NW = NC * NS
N_TOK = 64 * 32 * 32
CHUNK = 128
PER_W = N_TOK // NW
N_CHUNKS = PER_W // CHUNK


def _rsqrt_nr(xv):
    iv = lax.bitcast_convert_type(xv, jnp.int32)
    iv = jnp.int32(0x5F3759DF) - (iv >> 1)
    y = lax.bitcast_convert_type(iv, jnp.float32)
    for _ in range(3):
        y = y * (1.5 - 0.5 * xv * y * y)
    return y


def _sc_embed_ln(ids_hbm, pids_hbm, wemb_hbm, pemb_hbm, lnw_hbm, lnb_hbm,
                 out_hbm, idw_v, idp_v, wrows_v, prows_v, obuf_v,
                 lnw_v, lnb_v, semw, semp, semo):
    wid = lax.axis_index("s") * NC + lax.axis_index("c")
    row0 = wid * N_CHUNKS
    base_w = wid * PER_W

    pltpu.sync_copy(lnw_hbm, lnw_v)
    pltpu.sync_copy(lnb_hbm, lnb_v)
    pltpu.sync_copy(ids_hbm.at[pl.ds(row0, N_CHUNKS)], idw_v)
    pltpu.sync_copy(pids_hbm.at[pl.ds(row0, N_CHUNKS)], idp_v)
    lnw = [lnw_v[pl.ds(L * j, L)] for j in range(NV)]
    lnb = [lnb_v[pl.ds(L * j, L)] for j in range(NV)]

    def one_token(t, wr, pr, ob):
        w = [wr[t, pl.ds(L * j, L)] + pr[t, pl.ds(L * j, L)]
             for j in range(NV)]
        s = ((w[0] + w[1]) + (w[2] + w[3])) + ((w[4] + w[5]) + (w[6] + w[7]))
        sq = (((w[0] * w[0] + w[1] * w[1]) + (w[2] * w[2] + w[3] * w[3]))
              + ((w[4] * w[4] + w[5] * w[5]) + (w[6] * w[6] + w[7] * w[7])))
        tot = jnp.sum(s)
        totsq = jnp.sum(sq)
        u = tot * (1.0 / HIDDEN)
        var = totsq * (1.0 / HIDDEN) - u * u
        xv = jnp.full((L,), var + EPS, jnp.float32)
        r = _rsqrt_nr(xv)
        uv = jnp.full((L,), u, jnp.float32)
        for j in range(NV):
            ob[t, pl.ds(L * j, L)] = (w[j] - uv) * r * lnw[j] + lnb[j]

    def make_chunk_body(wr, pr, ob):
        def chunk_body(t2, carry):
            one_token(2 * t2, wr, pr, ob)
            one_token(2 * t2 + 1, wr, pr, ob)
            return carry
        return chunk_body

    def fire_gathers(c, b):
        cw = pltpu.async_copy(wemb_hbm.at[idw_v.at[c]], wrows_v.at[b], semw.at[b])
        cp = pltpu.async_copy(pemb_hbm.at[idp_v.at[c]], prows_v.at[b], semp.at[b])
        return cw, cp

    gath = [None, None]
    outc = [None, None]
    gath[0] = fire_gathers(0, 0)
    for c in range(N_CHUNKS):
        b = c % 2
        if c + 1 < N_CHUNKS:
            gath[1 - b] = fire_gathers(c + 1, 1 - b)
        cw, cp = gath[b]
        cw.wait()
        cp.wait()
        if outc[b] is not None:
            outc[b].wait()
        lax.fori_loop(0, CHUNK // 2,
                      make_chunk_body(wrows_v.at[b], prows_v.at[b],
                                      obuf_v.at[b]), 0)
        co = pltpu.async_copy(obuf_v.at[b],
                              out_hbm.at[pl.ds(base_w + c * CHUNK, CHUNK)],
                              semo.at[b])
        outc[b] = co
    outc[0].wait()
    outc[1].wait()


@jax.jit
def kernel(input_ids, position_ids, word_emb, pos_emb, ln_w, ln_b):
    ids = input_ids.reshape(N_TOK // CHUNK, CHUNK)
    pids = position_ids.reshape(N_TOK // CHUNK, CHUNK)
    mesh = plsc.VectorSubcoreMesh(core_axis_name="c", subcore_axis_name="s")
    k = functools.partial(
        pl.kernel,
        mesh=mesh,
        compiler_params=pltpu.CompilerParams(needs_layout_passes=False),
        out_type=jax.ShapeDtypeStruct((N_TOK, HIDDEN), jnp.float32),
        scratch_types=[
            pltpu.VMEM((N_CHUNKS, CHUNK), jnp.int32),
            pltpu.VMEM((N_CHUNKS, CHUNK), jnp.int32),
            pltpu.VMEM((2, CHUNK, HIDDEN), jnp.float32),
            pltpu.VMEM((2, CHUNK, HIDDEN), jnp.float32),
            pltpu.VMEM((2, CHUNK, HIDDEN), jnp.float32),
            pltpu.VMEM((HIDDEN,), jnp.float32),
            pltpu.VMEM((HIDDEN,), jnp.float32),
            pltpu.SemaphoreType.DMA((2,)),
            pltpu.SemaphoreType.DMA((2,)),
            pltpu.SemaphoreType.DMA((2,)),
        ],
    )(_sc_embed_ln)
    out = k(ids, pids, word_emb, pos_emb, ln_w, ln_b)
    return out.reshape(input_ids.shape + (HIDDEN,))
